# baseline (speedup 1.0000x reference)
import jax
import jax.numpy as jnp
from jax import lax
from jax.experimental import pallas as pl
from jax.experimental.pallas import tpu as pltpu
from jax.experimental.pallas import tpu_sc as plsc

N_NODES = 100000
D = 128
CHUNK = 80
NCHUNKS = N_NODES // CHUNK
NW = 32
TRIPS = -(-NCHUNKS // NW)


def _emb_body(elem, arom, chg, hct, We, Wa, Wc, Wh, out,
              idx_e, idx_a, idx_c, idx_h,
              rows_e, rows_a, rows_c, rows_h, sem):
    wid = lax.axis_index("s") * 2 + lax.axis_index("c")

    def trip(t, carry):
        c = t * NW + wid

        @pl.when(c < NCHUNKS)
        def _():
            base = c * CHUNK
            pltpu.sync_copy(elem.at[pl.ds(base, CHUNK)], idx_e)
            pltpu.sync_copy(arom.at[pl.ds(base, CHUNK)], idx_a)
            pltpu.sync_copy(chg.at[pl.ds(base, CHUNK)], idx_c)
            pltpu.sync_copy(hct.at[pl.ds(base, CHUNK)], idx_h)
            g0 = pltpu.async_copy(We.at[idx_e], rows_e, sem)
            g1 = pltpu.async_copy(Wa.at[idx_a], rows_a, sem)
            g2 = pltpu.async_copy(Wc.at[idx_c], rows_c, sem)
            g3 = pltpu.async_copy(Wh.at[idx_h], rows_h, sem)
            g0.wait()
            g1.wait()
            g2.wait()
            g3.wait()
            pltpu.sync_copy(rows_e, out.at[pl.ds(base, CHUNK), 0])
            pltpu.sync_copy(rows_a, out.at[pl.ds(base, CHUNK), 1])
            pltpu.sync_copy(rows_c, out.at[pl.ds(base, CHUNK), 2])
            pltpu.sync_copy(rows_h, out.at[pl.ds(base, CHUNK), 3])

        return carry

    lax.fori_loop(0, TRIPS, trip, None)


def kernel(element, aromatic, charge, hcount,
           W_element, W_aromatic, W_charge, W_hcount):
    mesh = plsc.VectorSubcoreMesh(core_axis_name="c", subcore_axis_name="s")
    f = pl.kernel(
        _emb_body,
        mesh=mesh,
        out_type=jax.ShapeDtypeStruct((N_NODES, 4, D), jnp.float32),
        scratch_types=[
            pltpu.VMEM((CHUNK,), jnp.int32),
            pltpu.VMEM((CHUNK,), jnp.int32),
            pltpu.VMEM((CHUNK,), jnp.int32),
            pltpu.VMEM((CHUNK,), jnp.int32),
            pltpu.VMEM((CHUNK, D), jnp.float32),
            pltpu.VMEM((CHUNK, D), jnp.float32),
            pltpu.VMEM((CHUNK, D), jnp.float32),
            pltpu.VMEM((CHUNK, D), jnp.float32),
            pltpu.SemaphoreType.DMA,
        ],
    )
    out = f(element, aromatic, charge, hcount,
            W_element, W_aromatic, W_charge, W_hcount)
    return out.reshape(N_NODES, 4 * D)
